# Initial kernel scaffold; baseline (speedup 1.0000x reference)
import jax
import jax.numpy as jnp
from jax.experimental import pallas as pl


def kernel(x, table):
    raise NotImplementedError("write your pallas kernel here")


# speedup vs baseline: 1.3621x; 1.3621x over previous
import functools
import math

import jax
import jax.numpy as jnp
from jax import lax
from jax.experimental import pallas as pl
from jax.experimental.pallas import tpu as pltpu
from jax.experimental.pallas import tpu_sc as plsc

EMB_D = 128
NUM_EMB = 22
PAD_IDX = 21
SCALE = math.sqrt(float(EMB_D))

NUM_CORES = 2
NUM_SUBCORES = 16
NUM_WORKERS = NUM_CORES * NUM_SUBCORES

TOTAL = 4096 * 200
PER_WORKER = TOTAL // NUM_WORKERS
CHUNK = 128
NUM_CHUNKS = PER_WORKER // CHUNK


def _scale_body(tab_ref, out_ref):
    rows = lax.broadcasted_iota(jnp.int32, (NUM_EMB, EMB_D), 0)
    scale = jnp.where(rows == PAD_IDX, 0.0, SCALE).astype(jnp.float32)
    out_ref[...] = tab_ref[...] * scale


_scale_table = pl.pallas_call(
    _scale_body,
    out_shape=jax.ShapeDtypeStruct((NUM_EMB, EMB_D), jnp.float32),
)

_sc_mesh = plsc.VectorSubcoreMesh(core_axis_name="c", subcore_axis_name="s")


@functools.partial(
    pl.kernel,
    mesh=_sc_mesh,
    out_type=jax.ShapeDtypeStruct((TOTAL, EMB_D), jnp.float32),
    scratch_types=[
        pltpu.VMEM((CHUNK,), jnp.int32),
        pltpu.VMEM((CHUNK, EMB_D), jnp.float32),
        pltpu.SemaphoreType.DMA,
    ],
)
def _gather_kernel(table_hbm, idx_hbm, out_hbm, idx_v, rows_v, sem):
    wid = lax.axis_index("s") * NUM_CORES + lax.axis_index("c")
    base = wid * PER_WORKER

    def body(i, carry):
        off = base + i * CHUNK
        pltpu.sync_copy(idx_hbm.at[pl.ds(off, CHUNK)], idx_v)
        pltpu.async_copy(table_hbm.at[idx_v], rows_v, sem).wait()
        pltpu.sync_copy(rows_v, out_hbm.at[pl.ds(off, CHUNK)])
        return carry

    lax.fori_loop(0, NUM_CHUNKS, body, 0)


def kernel(x, table):
    tab = _scale_table(table)
    idx = x.reshape(-1).astype(jnp.int32)
    out = _gather_kernel(tab, idx)
    return out.reshape(x.shape[0], x.shape[1], EMB_D)


# speedup vs baseline: 1.3783x; 1.0119x over previous
import functools
import math

import jax
import jax.numpy as jnp
from jax import lax
from jax.experimental import pallas as pl
from jax.experimental.pallas import tpu as pltpu
from jax.experimental.pallas import tpu_sc as plsc

EMB_D = 128
NUM_EMB = 22
PAD_IDX = 21
SCALE = math.sqrt(float(EMB_D))

NUM_CORES = 2
NUM_SUBCORES = 16
NUM_WORKERS = NUM_CORES * NUM_SUBCORES

TOTAL = 4096 * 200
PER_WORKER = TOTAL // NUM_WORKERS
CHUNK = 128
NUM_CHUNKS = PER_WORKER // CHUNK
NUM_PAIRS = NUM_CHUNKS // 2


def _scale_body(tab_ref, out_ref):
    rows = lax.broadcasted_iota(jnp.int32, (NUM_EMB, EMB_D), 0)
    scale = jnp.where(rows == PAD_IDX, 0.0, SCALE).astype(jnp.float32)
    out_ref[...] = tab_ref[...] * scale


_scale_table = pl.pallas_call(
    _scale_body,
    out_shape=jax.ShapeDtypeStruct((NUM_EMB, EMB_D), jnp.float32),
)

_sc_mesh = plsc.VectorSubcoreMesh(core_axis_name="c", subcore_axis_name="s")


@functools.partial(
    pl.kernel,
    mesh=_sc_mesh,
    out_type=jax.ShapeDtypeStruct((TOTAL, EMB_D), jnp.float32),
    scratch_types=[
        pltpu.VMEM((NUM_CHUNKS, CHUNK), jnp.int32),
        pltpu.VMEM((4, CHUNK, EMB_D), jnp.float32),
        pltpu.SemaphoreType.DMA,
        pltpu.SemaphoreType.DMA,
    ],
)
def _gather_kernel(table_hbm, idx_hbm, out_hbm, idx_v, rows_v, gsem, ssem):
    wid = lax.axis_index("s") * NUM_CORES + lax.axis_index("c")
    base = wid * PER_WORKER

    def fire_pair(p, b0, b1):
        pltpu.async_copy(table_hbm.at[idx_v.at[2 * p]], rows_v.at[b0], gsem)
        pltpu.async_copy(table_hbm.at[idx_v.at[2 * p + 1]], rows_v.at[b1], gsem)

    def drain_gathers(b0, b1):
        pltpu.make_async_copy(out_hbm.at[pl.ds(0, CHUNK)], rows_v.at[b0], gsem).wait()
        pltpu.make_async_copy(out_hbm.at[pl.ds(0, CHUNK)], rows_v.at[b1], gsem).wait()

    def scatter_pair(p, b0, b1):
        off = base + p * (2 * CHUNK)
        pltpu.async_copy(rows_v.at[b0], out_hbm.at[pl.ds(off, CHUNK)], ssem)
        pltpu.async_copy(rows_v.at[b1], out_hbm.at[pl.ds(off + CHUNK, CHUNK)], ssem)

    def drain_scatters(b0, b1):
        pltpu.make_async_copy(rows_v.at[b0], out_hbm.at[pl.ds(0, CHUNK)], ssem).wait()
        pltpu.make_async_copy(rows_v.at[b1], out_hbm.at[pl.ds(0, CHUNK)], ssem).wait()

    def process_pair(p, my0, my1, fire_next, nxt0, nxt1):
        if fire_next:
            fire_pair(p + 1, nxt0, nxt1)
        drain_gathers(my0, my1)
        scatter_pair(p, my0, my1)
        drain_scatters(my0, my1)

    pltpu.sync_copy(idx_hbm.at[pl.ds(wid * NUM_CHUNKS, NUM_CHUNKS)], idx_v)
    fire_pair(0, 0, 1)

    def body(c, carry):
        process_pair(2 * c, 0, 1, True, 2, 3)
        process_pair(2 * c + 1, 2, 3, True, 0, 1)
        return carry

    lax.fori_loop(0, NUM_PAIRS // 2 - 1, body, 0)
    process_pair(NUM_PAIRS - 2, 0, 1, True, 2, 3)
    process_pair(NUM_PAIRS - 1, 2, 3, False, 0, 0)


def kernel(x, table):
    tab = _scale_table(table)
    idx = x.reshape(NUM_WORKERS * NUM_CHUNKS, CHUNK).astype(jnp.int32)
    out = _gather_kernel(tab, idx)
    return out.reshape(x.shape[0], x.shape[1], EMB_D)


# speedup vs baseline: 15.8024x; 11.4656x over previous
import functools
import math

import jax
import jax.numpy as jnp
from jax import lax
from jax.experimental import pallas as pl
from jax.experimental.pallas import tpu as pltpu
from jax.experimental.pallas import tpu_sc as plsc

EMB_D = 128
NUM_EMB = 22
PAD_IDX = 21
SCALE = math.sqrt(float(EMB_D))

NUM_CORES = 2
NUM_SUBCORES = 16
NUM_WORKERS = NUM_CORES * NUM_SUBCORES

TOTAL = 4096 * 200
PER_WORKER = TOTAL // NUM_WORKERS
CHUNK = 128
NUM_CHUNKS = PER_WORKER // CHUNK
NUM_PAIRS = NUM_CHUNKS // 2


def _scale_body(tab_ref, out_ref):
    rows = lax.broadcasted_iota(jnp.int32, (NUM_EMB, EMB_D), 0)
    scale = jnp.where(rows == PAD_IDX, 0.0, SCALE).astype(jnp.float32)
    out_ref[...] = tab_ref[...] * scale


_scale_table = pl.pallas_call(
    _scale_body,
    out_shape=jax.ShapeDtypeStruct((NUM_EMB, EMB_D), jnp.float32),
)

_sc_mesh = plsc.VectorSubcoreMesh(core_axis_name="c", subcore_axis_name="s")


@functools.partial(
    pl.kernel,
    mesh=_sc_mesh,
    out_type=jax.ShapeDtypeStruct((TOTAL, EMB_D), jnp.float32),
    scratch_types=[
        pltpu.VMEM((NUM_CHUNKS, CHUNK), jnp.int32),
        pltpu.VMEM((4, CHUNK, EMB_D), jnp.float32),
        pltpu.VMEM_SHARED((NUM_EMB, EMB_D), jnp.float32),
        pltpu.SemaphoreType.DMA,
        pltpu.SemaphoreType.DMA,
    ],
)
def _gather_kernel(table_hbm, idx_hbm, out_hbm, idx_v, rows_v, tab_sh, gsem, ssem):
    wid = lax.axis_index("s") * NUM_CORES + lax.axis_index("c")
    base = wid * PER_WORKER

    @pl.when(lax.axis_index("s") == 0)
    def _stage_table():
        pltpu.sync_copy(table_hbm, tab_sh)

    plsc.subcore_barrier()

    def fire_pair(p, b0, b1):
        pltpu.async_copy(tab_sh.at[idx_v.at[2 * p]], rows_v.at[b0], gsem)
        pltpu.async_copy(tab_sh.at[idx_v.at[2 * p + 1]], rows_v.at[b1], gsem)

    def drain_gathers(b0, b1):
        pltpu.make_async_copy(out_hbm.at[pl.ds(0, CHUNK)], rows_v.at[b0], gsem).wait()
        pltpu.make_async_copy(out_hbm.at[pl.ds(0, CHUNK)], rows_v.at[b1], gsem).wait()

    def scatter_pair(p, b0, b1):
        off = base + p * (2 * CHUNK)
        pltpu.async_copy(rows_v.at[b0], out_hbm.at[pl.ds(off, CHUNK)], ssem)
        pltpu.async_copy(rows_v.at[b1], out_hbm.at[pl.ds(off + CHUNK, CHUNK)], ssem)

    def drain_scatters(b0, b1):
        pltpu.make_async_copy(rows_v.at[b0], out_hbm.at[pl.ds(0, CHUNK)], ssem).wait()
        pltpu.make_async_copy(rows_v.at[b1], out_hbm.at[pl.ds(0, CHUNK)], ssem).wait()

    def process_pair(p, my0, my1, fire_next, nxt0, nxt1):
        if fire_next:
            fire_pair(p + 1, nxt0, nxt1)
        drain_gathers(my0, my1)
        scatter_pair(p, my0, my1)
        drain_scatters(my0, my1)

    pltpu.sync_copy(idx_hbm.at[pl.ds(wid * NUM_CHUNKS, NUM_CHUNKS)], idx_v)
    fire_pair(0, 0, 1)

    def body(c, carry):
        process_pair(2 * c, 0, 1, True, 2, 3)
        process_pair(2 * c + 1, 2, 3, True, 0, 1)
        return carry

    lax.fori_loop(0, NUM_PAIRS // 2 - 1, body, 0)
    process_pair(NUM_PAIRS - 2, 0, 1, True, 2, 3)
    process_pair(NUM_PAIRS - 1, 2, 3, False, 0, 0)


def kernel(x, table):
    tab = _scale_table(table)
    idx = x.reshape(NUM_WORKERS * NUM_CHUNKS, CHUNK).astype(jnp.int32)
    out = _gather_kernel(tab, idx)
    return out.reshape(x.shape[0], x.shape[1], EMB_D)


# speedup vs baseline: 15.8572x; 1.0035x over previous
import functools
import math

import jax
import jax.numpy as jnp
from jax import lax
from jax.experimental import pallas as pl
from jax.experimental.pallas import tpu as pltpu
from jax.experimental.pallas import tpu_sc as plsc

EMB_D = 128
NUM_EMB = 22
PAD_IDX = 21
SCALE = math.sqrt(float(EMB_D))

NUM_CORES = 2
NUM_SUBCORES = 16
NUM_WORKERS = NUM_CORES * NUM_SUBCORES

TOTAL = 4096 * 200
PER_WORKER = TOTAL // NUM_WORKERS
CHUNK = 128
NUM_CHUNKS = PER_WORKER // CHUNK
NUM_PAIRS = NUM_CHUNKS // 2
LANES = 16

_sc_mesh = plsc.VectorSubcoreMesh(core_axis_name="c", subcore_axis_name="s")


@functools.partial(
    pl.kernel,
    mesh=_sc_mesh,
    out_type=jax.ShapeDtypeStruct((TOTAL, EMB_D), jnp.float32),
    scratch_types=[
        pltpu.VMEM((NUM_CHUNKS, CHUNK), jnp.int32),
        pltpu.VMEM((4, CHUNK, EMB_D), jnp.float32),
        pltpu.VMEM((NUM_EMB, EMB_D), jnp.float32),
        pltpu.VMEM_SHARED((NUM_EMB, EMB_D), jnp.float32),
        pltpu.SemaphoreType.DMA,
        pltpu.SemaphoreType.DMA,
    ],
)
def _emb_kernel(table_hbm, idx_hbm, out_hbm, idx_v, rows_v, tab_v, tab_sh, gsem, ssem):
    wid = lax.axis_index("s") * NUM_CORES + lax.axis_index("c")
    base = wid * PER_WORKER

    @pl.when(lax.axis_index("s") == 0)
    def _stage_table():
        pltpu.sync_copy(table_hbm, tab_v)
        for r in range(NUM_EMB):
            for k in range(EMB_D // LANES):
                sl = pl.ds(k * LANES, LANES)
                if r == PAD_IDX:
                    tab_v[r, sl] = jnp.zeros((LANES,), jnp.float32)
                else:
                    tab_v[r, sl] = tab_v[r, sl] * SCALE
        pltpu.sync_copy(tab_v, tab_sh)

    plsc.subcore_barrier()

    def fire_pair(p, b0, b1):
        pltpu.async_copy(tab_sh.at[idx_v.at[2 * p]], rows_v.at[b0], gsem)
        pltpu.async_copy(tab_sh.at[idx_v.at[2 * p + 1]], rows_v.at[b1], gsem)

    def drain_gathers(b0, b1):
        pltpu.make_async_copy(out_hbm.at[pl.ds(0, CHUNK)], rows_v.at[b0], gsem).wait()
        pltpu.make_async_copy(out_hbm.at[pl.ds(0, CHUNK)], rows_v.at[b1], gsem).wait()

    def scatter_pair(p, b0, b1):
        off = base + p * (2 * CHUNK)
        pltpu.async_copy(rows_v.at[b0], out_hbm.at[pl.ds(off, CHUNK)], ssem)
        pltpu.async_copy(rows_v.at[b1], out_hbm.at[pl.ds(off + CHUNK, CHUNK)], ssem)

    def drain_scatters(b0, b1):
        pltpu.make_async_copy(rows_v.at[b0], out_hbm.at[pl.ds(0, CHUNK)], ssem).wait()
        pltpu.make_async_copy(rows_v.at[b1], out_hbm.at[pl.ds(0, CHUNK)], ssem).wait()

    def process_pair(p, my0, my1, fire_next, nxt0, nxt1):
        if fire_next:
            fire_pair(p + 1, nxt0, nxt1)
        drain_gathers(my0, my1)
        scatter_pair(p, my0, my1)
        drain_scatters(my0, my1)

    pltpu.sync_copy(idx_hbm.at[pl.ds(wid * NUM_CHUNKS, NUM_CHUNKS)], idx_v)
    fire_pair(0, 0, 1)

    def body(c, carry):
        process_pair(2 * c, 0, 1, True, 2, 3)
        process_pair(2 * c + 1, 2, 3, True, 0, 1)
        return carry

    lax.fori_loop(0, NUM_PAIRS // 2 - 1, body, 0)
    process_pair(NUM_PAIRS - 2, 0, 1, True, 2, 3)
    process_pair(NUM_PAIRS - 1, 2, 3, False, 0, 0)


def kernel(x, table):
    idx = x.reshape(NUM_WORKERS * NUM_CHUNKS, CHUNK).astype(jnp.int32)
    out = _emb_kernel(table, idx)
    return out.reshape(x.shape[0], x.shape[1], EMB_D)
